# Initial kernel scaffold; baseline (speedup 1.0000x reference)
import jax
import jax.numpy as jnp
from jax.experimental import pallas as pl


def kernel(anchor, offset, score):
    raise NotImplementedError("write your pallas kernel here")


# speedup vs baseline: 274.6068x; 274.6068x over previous
import jax
import jax.numpy as jnp
from jax import lax
from jax.experimental import pallas as pl
from jax.experimental.pallas import tpu as pltpu

N_PRE = 12000
N_POST = 2000
TH = 0.7
B = 128
NB = 94
PAD = NB * B
F32 = jnp.float32


def _decode(aref, oref, bref):
    a0, a1, a2, a3 = aref[0], aref[1], aref[2], aref[3]
    cy = (a0 + a2) / 2
    cx = (a1 + a3) / 2
    h = a2 - a0
    w = a3 - a1
    dy, dx = oref[0], oref[1]
    dh = 0.5 * jnp.exp(oref[2])
    dw = 0.5 * jnp.exp(oref[3])
    y1 = jnp.clip(cy + h * (dy - dh), 0.0, 256.0)
    x1 = jnp.clip(cx + w * (dx - dw), 0.0, 256.0)
    y2 = jnp.clip(cy + h * (dy + dh), 0.0, 256.0)
    x2 = jnp.clip(cx + w * (dx + dw), 0.0, 256.0)
    bref[0] = y1
    bref[1] = x1
    bref[2] = y2
    bref[3] = x2
    bref[4] = (y2 - y1) * (x2 - x1)


def _nms_body(ancA, offA, out, bA, dA, s_scr):
    _decode(ancA, offA, bA)

    iotaL = lax.broadcasted_iota(jnp.int32, (1, B), 1)
    iotaS = lax.broadcasted_iota(jnp.int32, (B, 1), 0)
    eye = (lax.broadcasted_iota(jnp.int32, (B, B), 0)
           == lax.broadcasted_iota(jnp.int32, (B, B), 1)).astype(F32)

    def block_step(k, count):
        def fast(cnt):
            for c in range(4):
                out[c, pl.ds(k, 1), :] = jnp.zeros((1, B), F32)
            return cnt

        def slow(cnt):
            rows = [bA[c, pl.ds(k, 1), :] for c in range(5)]
            stacked = jnp.concatenate(
                [jnp.broadcast_to(r, (B, B)) for r in rows], axis=1)
            cbs = lax.dot_general(stacked, eye, (((0,), (0,)), ((), ())),
                                  precision=lax.Precision.HIGHEST,
                                  preferred_element_type=F32)
            CB = [cbs[c * B:(c + 1) * B, :] for c in range(5)]

            def pbody(j, sup):
                cj = [dA[c, pl.ds(j, 1), :] for c in range(5)]
                tly = jnp.maximum(CB[0], cj[0])
                tlx = jnp.maximum(CB[1], cj[1])
                bry = jnp.minimum(CB[2], cj[2])
                brx = jnp.minimum(CB[3], cj[3])
                cap = jnp.where((tly < bry) & (tlx < brx),
                                (bry - tly) * (brx - tlx), 0.0)
                iou = cap / ((CB[4] + cj[4]) - cap)
                hit = jnp.max((iou >= TH).astype(F32), axis=1, keepdims=True)
                return jnp.maximum(sup, hit)

            sup_col = lax.fori_loop(0, k, pbody, jnp.zeros((B, 1), F32))

            tly = jnp.maximum(CB[0], rows[0])
            tlx = jnp.maximum(CB[1], rows[1])
            bry = jnp.minimum(CB[2], rows[2])
            brx = jnp.minimum(CB[3], rows[3])
            cap = jnp.where((tly < bry) & (tlx < brx),
                            (bry - tly) * (brx - tlx), 0.0)
            iou = cap / ((CB[4] + rows[4]) - cap)
            s_scr[...] = (iou >= TH).astype(F32)

            valid_row = ((k * B + iotaL) < N_PRE).astype(F32)

            def ibody(i, st):
                keep_row, kf_row, c0 = st
                cur = (iotaL == i).astype(F32)
                curS = (iotaS == i).astype(F32)
                srow = s_scr[pl.ds(i, 1), :]
                within = jnp.max(srow * keep_row)
                supi = jnp.max(sup_col * curS)
                vali = jnp.max(valid_row * cur)
                keep_i = jnp.where(within + supi > 0.0, 0.0, 1.0) * vali
                c1 = c0 + keep_i.astype(jnp.int32)
                kf_i = keep_i * (c1 <= N_POST).astype(F32)
                return (keep_row + cur * keep_i, kf_row + cur * kf_i, c1)

            keep_row, kf_row, c_new = lax.fori_loop(
                0, B, ibody,
                (jnp.zeros((1, B), F32), jnp.zeros((1, B), F32), cnt))

            for c in range(5):
                dA[c, pl.ds(k, 1), :] = rows[c] * keep_row
            for c in range(4):
                out[c, pl.ds(k, 1), :] = rows[c] * kf_row
            return c_new

        return lax.cond(count >= N_POST, fast, slow, count)

    lax.fori_loop(0, NB, block_step, jnp.int32(0))


def kernel(anchor, offset, score):
    _, order = jax.lax.top_k(score[:, 1], N_PRE)
    anc = jnp.zeros((PAD, 4), F32).at[:N_PRE].set(anchor[order])
    off = jnp.zeros((PAD, 4), F32).at[:N_PRE].set(offset[order])
    ancA = anc.T.reshape(4, NB, B)
    offA = off.T.reshape(4, NB, B)
    out_t = pl.pallas_call(
        _nms_body,
        out_shape=jax.ShapeDtypeStruct((4, NB, B), F32),
        scratch_shapes=[
            pltpu.VMEM((5, NB, B), F32),
            pltpu.VMEM((5, NB, B), F32),
            pltpu.VMEM((B, B), F32),
        ],
    )(ancA, offA)
    return out_t.reshape(4, PAD).T[:N_PRE]
